# baseline (speedup 1.0000x reference)
import functools

import jax
import jax.numpy as jnp
from jax import lax
from jax.experimental import pallas as pl
from jax.experimental.pallas import tpu as pltpu
from jax.experimental.pallas import tpu_sc as plsc


def _transform_body(emb_ref, w_ref, b_ref, out_ref):
    h = jnp.maximum(emb_ref[...], 0.0)
    out_ref[...] = (
        lax.dot_general(
            h, w_ref[...], (((1,), (1,)), ((), ())),
            preferred_element_type=jnp.float32,
        )
        + b_ref[...]
    )


def _transform_table(emb, W, b2d, blk):
    n_rows, hidden = emb.shape
    out_dim = W.shape[0]
    grid = (n_rows // blk,)
    return pl.pallas_call(
        _transform_body,
        grid=grid,
        in_specs=[
            pl.BlockSpec((blk, hidden), lambda i: (i, 0)),
            pl.BlockSpec((out_dim, hidden), lambda i: (0, 0)),
            pl.BlockSpec((1, out_dim), lambda i: (0, 0)),
        ],
        out_specs=pl.BlockSpec((blk, out_dim), lambda i: (i, 0)),
        out_shape=jax.ShapeDtypeStruct((n_rows, out_dim), jnp.float32),
        compiler_params=pltpu.CompilerParams(
            dimension_semantics=("arbitrary",),
        ),
    )(emb, W, b2d)


_NC = 2
_NS = 16
_NW = _NC * _NS
_CH = 128
_NBUF = 4


def _make_gather(n_idx, out_dim):
    n_chunks = n_idx // (_NW * _CH)
    mesh = plsc.VectorSubcoreMesh(core_axis_name="c", subcore_axis_name="s")

    @functools.partial(
        pl.kernel,
        mesh=mesh,
        out_type=jax.ShapeDtypeStruct((n_idx, out_dim), jnp.float32),
        scratch_types=[
            pltpu.VMEM((n_chunks, _CH), jnp.int32),
            pltpu.VMEM((_NBUF, _CH, out_dim), jnp.float32),
            pltpu.SemaphoreType.DMA,
            pltpu.SemaphoreType.DMA,
        ],
        compiler_params=pltpu.CompilerParams(use_tc_tiling_on_sc=False),
    )
    def gather_k(table_hbm, idx_hbm, out_hbm, idx_v, rows_v, gsem, osem):
        wid = lax.axis_index("s") * _NC + lax.axis_index("c")
        rows_per_w = n_chunks * _CH
        base = wid * rows_per_w
        pltpu.sync_copy(idx_hbm.at[wid], idx_v)

        def issue(j, buf):
            return pltpu.async_copy(
                table_hbm.at[idx_v.at[j]], rows_v.at[buf], gsem
            )

        for j in range(_NBUF):
            issue(j, j)

        def step(j, carry):
            buf = lax.rem(j, _NBUF)
            pltpu.make_async_copy(
                table_hbm.at[idx_v.at[j]], rows_v.at[buf], gsem
            ).wait()
            out_copy = pltpu.async_copy(
                rows_v.at[buf], out_hbm.at[pl.ds(base + j * _CH, _CH)], osem
            )
            out_copy.wait()
            @pl.when(j + _NBUF < n_chunks)
            def _():
                issue(j + _NBUF, buf)
            return carry

        lax.fori_loop(0, n_chunks, step, 0, unroll=False)

    return gather_k


def kernel(X, emb, W, b):
    B, L = X.shape
    n_rows, hidden = emb.shape
    out_dim = W.shape[0]
    n_idx = B * L
    n_chunks = n_idx // (_NW * _CH)
    assert n_idx == _NW * n_chunks * _CH

    idx = X.reshape(_NW, n_chunks, _CH).astype(jnp.int32)
    gathered = _make_gather(n_idx, hidden)(emb, idx)
    out = _transform_table(gathered, W, b.reshape(1, out_dim), blk=8192)
    return out.reshape(B, L, out_dim)
